# baseline (speedup 1.0000x reference)
import functools

import jax
import jax.numpy as jnp
from jax import lax
from jax.experimental import pallas as pl
from jax.experimental.pallas import tpu as pltpu
from jax.experimental.pallas import tpu_sc as plsc

N = 50000
E = 800000
G = 64
IN_DIM = 128
HID = 64

NP = 50176
ECH = 192
CPT = 264
EP = 16 * CPT * ECH
BLK = 12
NBLK = CPT // BLK
RPT = NP // 16
DEG_CPT = 224
DEG_BLK = 32
DEG_NBLK = DEG_CPT // DEG_BLK
EP_DEG = 32 * DEG_CPT * 128
ROWB = 3136
GRID = NP // ROWB
ROWB_P = 1024
GRID_P = NP // ROWB_P

_mesh_cache = []


def _sc_mesh():
    if not _mesh_cache:
        _mesh_cache.append(plsc.VectorSubcoreMesh(
            core_axis_name="c", subcore_axis_name="s",
            num_cores=2, num_subcores=16))
    return _mesh_cache[0]


def _deg_body(de_hbm, ones_hbm, zeros_hbm, out_hbm,
              didx, ones_v, zrow, ssem, acc_deg):
    c = lax.axis_index("c")
    s = lax.axis_index("s")
    wid = s * 2 + c

    pltpu.sync_copy(ones_hbm, ones_v)
    pltpu.sync_copy(zeros_hbm, zrow)
    pltpu.sync_copy(zrow, acc_deg.at[pl.ds(s * RPT, RPT)])
    plsc.subcore_barrier()

    base = wid * DEG_CPT

    @pl.loop(0, DEG_NBLK)
    def _block(b):
        row0 = base + b * DEG_BLK
        pltpu.sync_copy(de_hbm.at[pl.ds(row0, DEG_BLK)], didx)

        @pl.loop(0, DEG_BLK)
        def _fire(j):
            pltpu.async_copy(ones_v, acc_deg.at[didx.at[j]], ssem, add=True)

        @pl.loop(0, DEG_BLK)
        def _drain(j):
            pltpu.make_async_copy(ones_v, acc_deg.at[didx.at[0]], ssem).wait()

    plsc.subcore_barrier()
    pltpu.sync_copy(acc_deg.at[pl.ds(s * RPT, RPT)], zrow)
    pltpu.sync_copy(zrow, out_hbm.at[pl.ds(c * NP + s * RPT, RPT)])


def _edge_body(h2cat_hbm, se_hbm, de_hbm, zeros_hbm, out_hbm,
               sidx, didx, b0, b1, b2, b3,
               acc, g0, g1, g2, g3):
    c = lax.axis_index("c")
    s = lax.axis_index("s")
    cnp = c * NP
    bufs = (b0, b1, b2, b3)
    gsems = (g0, g1, g2, g3)

    pltpu.sync_copy(zeros_hbm, b0)

    @pl.loop(0, 24)
    def _z(k):
        pltpu.sync_copy(b0.at[pl.ds(0, 128)],
                        acc.at[pl.ds(s * RPT + k * 128, 128)])

    pltpu.sync_copy(b0.at[pl.ds(0, 64)],
                    acc.at[pl.ds(s * RPT + 3072, 64)])
    plsc.subcore_barrier()

    base = s * CPT

    @pl.loop(0, NBLK)
    def _block(b):
        row0 = base + b * BLK
        pltpu.sync_copy(se_hbm.at[pl.ds(row0, BLK)], sidx)
        pltpu.sync_copy(de_hbm.at[pl.ds(row0, BLK)], didx)

        @pl.loop(0, BLK)
        def _off(r):
            for k in range(ECH // 16):
                sl = pl.ds(k * 16, 16)
                sidx[r, sl] = sidx[r, sl] + cnp

        for p in range(4):
            pltpu.async_copy(h2cat_hbm.at[sidx.at[p]], bufs[p], gsems[p])

        @pl.loop(0, BLK - 4, step=4)
        def _grp(j):
            for p in range(4):
                jj = j + p
                pltpu.make_async_copy(
                    h2cat_hbm.at[sidx.at[0]], bufs[p], gsems[p]).wait()
                pltpu.sync_copy(bufs[p], acc.at[didx.at[jj]], add=True)
                pltpu.async_copy(
                    h2cat_hbm.at[sidx.at[jj + 4]], bufs[p], gsems[p])

        for p in range(4):
            pltpu.make_async_copy(
                h2cat_hbm.at[sidx.at[0]], bufs[p], gsems[p]).wait()
            pltpu.sync_copy(bufs[p], acc.at[didx.at[BLK - 4 + p]], add=True)

    plsc.subcore_barrier()

    @pl.loop(0, 24)
    def _o(k):
        pltpu.sync_copy(acc.at[pl.ds(s * RPT + k * 128, 128)],
                        b0.at[pl.ds(0, 128)])
        pltpu.sync_copy(b0.at[pl.ds(0, 128)],
                        out_hbm.at[pl.ds(cnp + s * RPT + k * 128, 128)])

    pltpu.sync_copy(acc.at[pl.ds(s * RPT + 3072, 64)], b0.at[pl.ds(0, 64)])
    pltpu.sync_copy(b0.at[pl.ds(0, 64)],
                    out_hbm.at[pl.ds(cnp + s * RPT + 3072, 64)])


def _dinv_of(degs):
    return lax.rsqrt(degs[0, 0, 0, :] + degs[1, 0, 0, :] + 1.0)


def _a1_body(x_ref, deg_ref, w_ref, o_ref):
    dinv = _dinv_of(deg_ref[...])
    h2 = jnp.dot(x_ref[...], w_ref[...],
                 preferred_element_type=jnp.float32) * dinv[:, None]
    o_ref[0] = h2[:, :32]
    o_ref[1] = h2[:, 32:]


def _a23_body(z_ref, st_ref, g_ref, be_ref, deg_ref, w_ref, o_ref):
    st = st_ref[...]
    m = st[0:1, :] / N
    v = st[1:2, :] / N - m * m
    xin = jax.nn.relu(g_ref[...] * (z_ref[...] - m) *
                      lax.rsqrt(v + 1e-5) + be_ref[...])
    dinv = _dinv_of(deg_ref[...])
    h2 = jnp.dot(xin, w_ref[...],
                 preferred_element_type=jnp.float32) * dinv[:, None]
    o_ref[0] = h2[:, :32]
    o_ref[1] = h2[:, 32:]


def _c_body(acc_ref, h2_ref, deg_ref, b_ref, z_ref, st_ref):
    i = pl.program_id(0)
    dinv = _dinv_of(deg_ref[...])[:, None]
    zl = dinv * (acc_ref[0] + h2_ref[0]) + b_ref[0, :32]
    zr = dinv * (acc_ref[1] + h2_ref[1]) + b_ref[0, 32:]
    z = jnp.concatenate([zl, zr], axis=1)
    z_ref[...] = z
    row0 = i * ROWB
    mask = (row0 + lax.broadcasted_iota(jnp.int32, (ROWB, 1), 0)) < N
    zm = jnp.where(mask, z, 0.0)

    @pl.when(i == 0)
    def _():
        st_ref[...] = jnp.zeros_like(st_ref)

    st_ref[0:1, :] += jnp.sum(zm, axis=0, keepdims=True)
    st_ref[1:2, :] += jnp.sum(zm * zm, axis=0, keepdims=True)


def _pool_body(z_ref, st_ref, g_ref, be_ref, batch_ref,
               w1_ref, bb1_ref, w2_ref, bb2_ref, w3_ref, bb3_ref,
               o_ref, ssum_s, smax_s, cnt_s):
    i = pl.program_id(0)
    st = st_ref[...]
    m = st[0:1, :] / N
    v = st[1:2, :] / N - m * m
    x = jax.nn.relu(g_ref[...] * (z_ref[...] - m) *
                    lax.rsqrt(v + 1e-5) + be_ref[...])
    b = batch_ref[0, 0, :]
    gid = lax.broadcasted_iota(jnp.int32, (ROWB_P, G), 1)
    onehot = jnp.where((b[:, None] == gid), 1.0, 0.0)

    @pl.when(i == 0)
    def _():
        ssum_s[...] = jnp.zeros_like(ssum_s)
        cnt_s[...] = jnp.zeros_like(cnt_s)
        smax_s[...] = jnp.full_like(smax_s, -jnp.inf)

    xz = jnp.where(b[:, None] < G, x, 0.0)
    ssum_s[...] += lax.dot_general(onehot, xz, (((0,), (0,)), ((), ())),
                                   preferred_element_type=jnp.float32)
    cnt_s[...] += lax.dot_general(onehot, jnp.ones((ROWB_P, 1), jnp.float32),
                                  (((0,), (0,)), ((), ())),
                                  preferred_element_type=jnp.float32)
    b_lo = batch_ref[0, 0, 0]
    b_hi = batch_ref[0, 0, ROWB_P - 1]
    for gg in range(G):
        @pl.when(jnp.logical_and(b_lo <= gg, gg <= b_hi))
        def _(gg=gg):
            xg = jnp.where(b[:, None] == gg, x, -jnp.inf)
            smax_s[gg:gg + 1, :] = jnp.maximum(
                smax_s[gg:gg + 1, :], jnp.max(xg, axis=0, keepdims=True))

    @pl.when(i == GRID_P - 1)
    def _():
        ssum = ssum_s[...]
        smean = ssum / jnp.maximum(cnt_s[...], 1.0)
        h = jnp.concatenate([smean, smax_s[...], ssum], axis=1)
        h = jax.nn.relu(jnp.dot(h, w1_ref[...],
                                preferred_element_type=jnp.float32)
                        + bb1_ref[...])
        h = jax.nn.relu(jnp.dot(h, w2_ref[...],
                                preferred_element_type=jnp.float32)
                        + bb2_ref[...])
        o_ref[...] = jnp.dot(h, w3_ref[...],
                             preferred_element_type=jnp.float32) + bb3_ref[...]


def _run_deg(de2d, ones_row, zeros_row):
    f = pl.kernel(
        _deg_body,
        out_type=jax.ShapeDtypeStruct((2 * NP,), jnp.float32),
        mesh=_sc_mesh(),
        compiler_params=pltpu.CompilerParams(use_tc_tiling_on_sc=False),
        scratch_types=[
            pltpu.VMEM((DEG_BLK, 128), jnp.int32),
            pltpu.VMEM((128,), jnp.float32),
            pltpu.VMEM((RPT,), jnp.float32),
            pltpu.SemaphoreType.DMA,
            pltpu.VMEM_SHARED((NP,), jnp.float32),
        ],
    )
    return f(de2d, ones_row, zeros_row)


def _run_edges(h2cat, se2d, de2d, zeros_blk):
    f = pl.kernel(
        _edge_body,
        out_type=jax.ShapeDtypeStruct((2 * NP, 32), jnp.float32),
        mesh=_sc_mesh(),
        compiler_params=pltpu.CompilerParams(use_tc_tiling_on_sc=False),
        scratch_types=[
            pltpu.VMEM((BLK, ECH), jnp.int32),
            pltpu.VMEM((BLK, ECH), jnp.int32),
            pltpu.VMEM((ECH, 32), jnp.float32),
            pltpu.VMEM((ECH, 32), jnp.float32),
            pltpu.VMEM((ECH, 32), jnp.float32),
            pltpu.VMEM((ECH, 32), jnp.float32),
            pltpu.VMEM_SHARED((NP, 32), jnp.float32),
            pltpu.SemaphoreType.DMA,
            pltpu.SemaphoreType.DMA,
            pltpu.SemaphoreType.DMA,
            pltpu.SemaphoreType.DMA,
        ],
    )
    return f(h2cat, se2d, de2d, zeros_blk)


def _run_a1(xp, deg2, W0):
    return pl.pallas_call(
        _a1_body,
        grid=(GRID,),
        in_specs=[
            pl.BlockSpec((ROWB, IN_DIM), lambda i: (i, 0)),
            pl.BlockSpec((2, 1, 1, ROWB), lambda i: (0, i, 0, 0)),
            pl.BlockSpec((IN_DIM, HID), lambda i: (0, 0)),
        ],
        out_specs=pl.BlockSpec((2, ROWB, 32), lambda i: (0, i, 0)),
        out_shape=jax.ShapeDtypeStruct((2, NP, 32), jnp.float32),
    )(xp, deg2, W0)


def _run_a23(z, st, g, be, deg2, W):
    return pl.pallas_call(
        _a23_body,
        grid=(GRID,),
        in_specs=[
            pl.BlockSpec((ROWB, HID), lambda i: (i, 0)),
            pl.BlockSpec((2, HID), lambda i: (0, 0)),
            pl.BlockSpec((1, HID), lambda i: (0, 0)),
            pl.BlockSpec((1, HID), lambda i: (0, 0)),
            pl.BlockSpec((2, 1, 1, ROWB), lambda i: (0, i, 0, 0)),
            pl.BlockSpec((HID, HID), lambda i: (0, 0)),
        ],
        out_specs=pl.BlockSpec((2, ROWB, 32), lambda i: (0, i, 0)),
        out_shape=jax.ShapeDtypeStruct((2, NP, 32), jnp.float32),
    )(z, st, g, be, deg2, W)


def _run_c(accs, h2pair, deg2, b):
    return pl.pallas_call(
        _c_body,
        grid=(GRID,),
        in_specs=[
            pl.BlockSpec((2, ROWB, 32), lambda i: (0, i, 0)),
            pl.BlockSpec((2, ROWB, 32), lambda i: (0, i, 0)),
            pl.BlockSpec((2, 1, 1, ROWB), lambda i: (0, i, 0, 0)),
            pl.BlockSpec((1, HID), lambda i: (0, 0)),
        ],
        out_specs=[
            pl.BlockSpec((ROWB, HID), lambda i: (i, 0)),
            pl.BlockSpec((2, HID), lambda i: (0, 0)),
        ],
        out_shape=[
            jax.ShapeDtypeStruct((NP, HID), jnp.float32),
            jax.ShapeDtypeStruct((2, HID), jnp.float32),
        ],
    )(accs, h2pair, deg2, b)


def _run_pool(z3, st3, g, be, batch3d, fcW1, fcb1, fcW2, fcb2, fcW3, fcb3):
    return pl.pallas_call(
        _pool_body,
        grid=(GRID_P,),
        in_specs=[
            pl.BlockSpec((ROWB_P, HID), lambda i: (i, 0)),
            pl.BlockSpec((2, HID), lambda i: (0, 0)),
            pl.BlockSpec((1, HID), lambda i: (0, 0)),
            pl.BlockSpec((1, HID), lambda i: (0, 0)),
            pl.BlockSpec((1, 1, ROWB_P), lambda i: (i, 0, 0)),
            pl.BlockSpec((3 * HID, HID), lambda i: (0, 0)),
            pl.BlockSpec((1, HID), lambda i: (0, 0)),
            pl.BlockSpec((HID, HID // 2), lambda i: (0, 0)),
            pl.BlockSpec((1, HID // 2), lambda i: (0, 0)),
            pl.BlockSpec((HID // 2, 1), lambda i: (0, 0)),
            pl.BlockSpec((1, 1), lambda i: (0, 0)),
        ],
        out_specs=pl.BlockSpec((G, 1), lambda i: (0, 0)),
        out_shape=jax.ShapeDtypeStruct((G, 1), jnp.float32),
        scratch_shapes=[
            pltpu.VMEM((G, HID), jnp.float32),
            pltpu.VMEM((G, HID), jnp.float32),
            pltpu.VMEM((G, 1), jnp.float32),
        ],
    )(z3, st3, g, be, batch3d, fcW1, fcb1, fcW2, fcb2, fcW3, fcb3)


def kernel(x, edge_index, batch, W0, b0, g0, be0, W1, b1, g1, be1,
           W2, b2, g2, be2, fcW1, fcb1, fcW2, fcb2, fcW3, fcb3):
    src = edge_index[0]
    dst = edge_index[1]

    padn = EP - E
    pad_src = (jnp.arange(padn, dtype=jnp.int32) * 67) % N
    pad_dst = N + (jnp.arange(padn, dtype=jnp.int32) % (NP - N))
    se2d = jnp.concatenate([src, pad_src]).reshape(EP // ECH, ECH)
    de2d = jnp.concatenate([dst, pad_dst]).reshape(EP // ECH, ECH)
    padd = EP_DEG - E
    pad_dst_deg = N + (jnp.arange(padd, dtype=jnp.int32) % (NP - N))
    ded2d = jnp.concatenate([dst, pad_dst_deg]).reshape(EP_DEG // 128, 128)
    xp = x
    batch3d = jnp.pad(batch, (0, NP - N), constant_values=G).reshape(
        GRID_P, 1, ROWB_P)
    ones_row = jnp.ones((128,), jnp.float32)
    zeros_row = jnp.zeros((RPT,), jnp.float32)
    zeros_blk = jnp.zeros((ECH, 32), jnp.float32)

    deg2 = _run_deg(ded2d, ones_row, zeros_row).reshape(2, GRID, 1, ROWB)

    h2p = _run_a1(xp, deg2, W0)
    accs = _run_edges(h2p.reshape(2 * NP, 32), se2d, de2d, zeros_blk)
    z, st = _run_c(accs.reshape(2, NP, 32), h2p, deg2, b0[None, :])

    h2p = _run_a23(z, st, g0[None, :], be0[None, :], deg2, W1)
    accs = _run_edges(h2p.reshape(2 * NP, 32), se2d, de2d, zeros_blk)
    z, st = _run_c(accs.reshape(2, NP, 32), h2p, deg2, b1[None, :])

    h2p = _run_a23(z, st, g1[None, :], be1[None, :], deg2, W2)
    accs = _run_edges(h2p.reshape(2 * NP, 32), se2d, de2d, zeros_blk)
    z, st = _run_c(accs.reshape(2, NP, 32), h2p, deg2, b2[None, :])

    out = _run_pool(z, st, g2[None, :], be2[None, :], batch3d,
                    fcW1, fcb1[None, :], fcW2, fcb2[None, :],
                    fcW3, fcb3[None, :])
    return out


# speedup vs baseline: 1.0376x; 1.0376x over previous
import functools

import jax
import jax.numpy as jnp
from jax import lax
from jax.experimental import pallas as pl
from jax.experimental.pallas import tpu as pltpu
from jax.experimental.pallas import tpu_sc as plsc

N = 50000
E = 800000
G = 64
IN_DIM = 128
HID = 64

NP = 50176
ECH = 192
CPT = 264
EP = 16 * CPT * ECH
BLK = 24
NBLK = CPT // BLK
RPT = NP // 16
DEG_CPT = 224
DEG_BLK = 32
DEG_NBLK = DEG_CPT // DEG_BLK
EP_DEG = 32 * DEG_CPT * 128
ROWB = 3136
GRID = NP // ROWB
ROWB_P = 1024
GRID_P = NP // ROWB_P

_mesh_cache = []


def _sc_mesh():
    if not _mesh_cache:
        _mesh_cache.append(plsc.VectorSubcoreMesh(
            core_axis_name="c", subcore_axis_name="s",
            num_cores=2, num_subcores=16))
    return _mesh_cache[0]


def _deg_body(de_hbm, ones_hbm, zeros_hbm, out_hbm,
              didx, ones_v, zrow, ssem, acc_deg):
    c = lax.axis_index("c")
    s = lax.axis_index("s")
    wid = s * 2 + c

    pltpu.sync_copy(ones_hbm, ones_v)
    pltpu.sync_copy(zeros_hbm, zrow)
    pltpu.sync_copy(zrow, acc_deg.at[pl.ds(s * RPT, RPT)])
    plsc.subcore_barrier()

    base = wid * DEG_CPT

    @pl.loop(0, DEG_NBLK)
    def _block(b):
        row0 = base + b * DEG_BLK
        pltpu.sync_copy(de_hbm.at[pl.ds(row0, DEG_BLK)], didx)

        @pl.loop(0, DEG_BLK)
        def _fire(j):
            pltpu.async_copy(ones_v, acc_deg.at[didx.at[j]], ssem, add=True)

        @pl.loop(0, DEG_BLK)
        def _drain(j):
            pltpu.make_async_copy(ones_v, acc_deg.at[didx.at[0]], ssem).wait()

    plsc.subcore_barrier()
    pltpu.sync_copy(acc_deg.at[pl.ds(s * RPT, RPT)], zrow)
    pltpu.sync_copy(zrow, out_hbm.at[pl.ds(c * NP + s * RPT, RPT)])


def _edge_body(h2cat_hbm, se_hbm, de_hbm, zeros_hbm, out_hbm,
               sidx, didx, b0, b1, b2,
               acc, g0, g1, g2):
    c = lax.axis_index("c")
    s = lax.axis_index("s")
    cnp = c * NP
    bufs = (b0, b1, b2)
    gsems = (g0, g1, g2)

    pltpu.sync_copy(zeros_hbm, b0)

    @pl.loop(0, 24)
    def _z(k):
        pltpu.sync_copy(b0.at[pl.ds(0, 128)],
                        acc.at[pl.ds(s * RPT + k * 128, 128)])

    pltpu.sync_copy(b0.at[pl.ds(0, 64)],
                    acc.at[pl.ds(s * RPT + 3072, 64)])
    plsc.subcore_barrier()

    base = s * CPT

    @pl.loop(0, NBLK)
    def _block(b):
        row0 = base + b * BLK
        pltpu.sync_copy(se_hbm.at[pl.ds(row0, BLK)], sidx)
        pltpu.sync_copy(de_hbm.at[pl.ds(row0, BLK)], didx)

        @pl.loop(0, BLK)
        def _off(r):
            for k in range(ECH // 16):
                sl = pl.ds(k * 16, 16)
                sidx[r, sl] = sidx[r, sl] + cnp

        for p in range(3):
            pltpu.async_copy(h2cat_hbm.at[sidx.at[p]], bufs[p], gsems[p])

        @pl.loop(0, BLK - 3, step=3)
        def _grp(j):
            for p in range(3):
                jj = j + p
                pltpu.make_async_copy(
                    h2cat_hbm.at[sidx.at[0]], bufs[p], gsems[p]).wait()
                pltpu.sync_copy(bufs[p], acc.at[didx.at[jj]], add=True)
                pltpu.async_copy(
                    h2cat_hbm.at[sidx.at[jj + 3]], bufs[p], gsems[p])

        for p in range(3):
            pltpu.make_async_copy(
                h2cat_hbm.at[sidx.at[0]], bufs[p], gsems[p]).wait()
            pltpu.sync_copy(bufs[p], acc.at[didx.at[BLK - 3 + p]], add=True)

    plsc.subcore_barrier()

    @pl.loop(0, 24)
    def _o(k):
        pltpu.sync_copy(acc.at[pl.ds(s * RPT + k * 128, 128)],
                        b0.at[pl.ds(0, 128)])
        pltpu.sync_copy(b0.at[pl.ds(0, 128)],
                        out_hbm.at[pl.ds(cnp + s * RPT + k * 128, 128)])

    pltpu.sync_copy(acc.at[pl.ds(s * RPT + 3072, 64)], b0.at[pl.ds(0, 64)])
    pltpu.sync_copy(b0.at[pl.ds(0, 64)],
                    out_hbm.at[pl.ds(cnp + s * RPT + 3072, 64)])


def _dinv_of(degs):
    return lax.rsqrt(degs[0, 0, 0, :] + degs[1, 0, 0, :] + 1.0)


def _a1_body(x_ref, deg_ref, w_ref, o_ref):
    dinv = _dinv_of(deg_ref[...])
    h2 = jnp.dot(x_ref[...], w_ref[...],
                 preferred_element_type=jnp.float32) * dinv[:, None]
    o_ref[0] = h2[:, :32]
    o_ref[1] = h2[:, 32:]


def _a23_body(z_ref, st_ref, g_ref, be_ref, deg_ref, w_ref, o_ref):
    st = st_ref[...]
    m = st[0:1, :] / N
    v = st[1:2, :] / N - m * m
    xin = jax.nn.relu(g_ref[...] * (z_ref[...] - m) *
                      lax.rsqrt(v + 1e-5) + be_ref[...])
    dinv = _dinv_of(deg_ref[...])
    h2 = jnp.dot(xin, w_ref[...],
                 preferred_element_type=jnp.float32) * dinv[:, None]
    o_ref[0] = h2[:, :32]
    o_ref[1] = h2[:, 32:]


def _c_body(acc_ref, h2_ref, deg_ref, b_ref, z_ref, st_ref):
    i = pl.program_id(0)
    dinv = _dinv_of(deg_ref[...])[:, None]
    zl = dinv * (acc_ref[0] + h2_ref[0]) + b_ref[0, :32]
    zr = dinv * (acc_ref[1] + h2_ref[1]) + b_ref[0, 32:]
    z = jnp.concatenate([zl, zr], axis=1)
    z_ref[...] = z
    row0 = i * ROWB
    mask = (row0 + lax.broadcasted_iota(jnp.int32, (ROWB, 1), 0)) < N
    zm = jnp.where(mask, z, 0.0)

    @pl.when(i == 0)
    def _():
        st_ref[...] = jnp.zeros_like(st_ref)

    st_ref[0:1, :] += jnp.sum(zm, axis=0, keepdims=True)
    st_ref[1:2, :] += jnp.sum(zm * zm, axis=0, keepdims=True)


def _pool_body(z_ref, st_ref, g_ref, be_ref, batch_ref,
               w1_ref, bb1_ref, w2_ref, bb2_ref, w3_ref, bb3_ref,
               o_ref, ssum_s, smax_s, cnt_s):
    i = pl.program_id(0)
    st = st_ref[...]
    m = st[0:1, :] / N
    v = st[1:2, :] / N - m * m
    x = jax.nn.relu(g_ref[...] * (z_ref[...] - m) *
                    lax.rsqrt(v + 1e-5) + be_ref[...])
    b = batch_ref[0, 0, :]
    gid = lax.broadcasted_iota(jnp.int32, (ROWB_P, G), 1)
    onehot = jnp.where((b[:, None] == gid), 1.0, 0.0)

    @pl.when(i == 0)
    def _():
        ssum_s[...] = jnp.zeros_like(ssum_s)
        cnt_s[...] = jnp.zeros_like(cnt_s)
        smax_s[...] = jnp.full_like(smax_s, -jnp.inf)

    xz = jnp.where(b[:, None] < G, x, 0.0)
    ssum_s[...] += lax.dot_general(onehot, xz, (((0,), (0,)), ((), ())),
                                   preferred_element_type=jnp.float32)
    cnt_s[...] += lax.dot_general(onehot, jnp.ones((ROWB_P, 1), jnp.float32),
                                  (((0,), (0,)), ((), ())),
                                  preferred_element_type=jnp.float32)
    b_lo = batch_ref[0, 0, 0]
    b_hi = batch_ref[0, 0, ROWB_P - 1]
    for gg in range(G):
        @pl.when(jnp.logical_and(b_lo <= gg, gg <= b_hi))
        def _(gg=gg):
            xg = jnp.where(b[:, None] == gg, x, -jnp.inf)
            smax_s[gg:gg + 1, :] = jnp.maximum(
                smax_s[gg:gg + 1, :], jnp.max(xg, axis=0, keepdims=True))

    @pl.when(i == GRID_P - 1)
    def _():
        ssum = ssum_s[...]
        smean = ssum / jnp.maximum(cnt_s[...], 1.0)
        h = jnp.concatenate([smean, smax_s[...], ssum], axis=1)
        h = jax.nn.relu(jnp.dot(h, w1_ref[...],
                                preferred_element_type=jnp.float32)
                        + bb1_ref[...])
        h = jax.nn.relu(jnp.dot(h, w2_ref[...],
                                preferred_element_type=jnp.float32)
                        + bb2_ref[...])
        o_ref[...] = jnp.dot(h, w3_ref[...],
                             preferred_element_type=jnp.float32) + bb3_ref[...]


def _run_deg(de2d, ones_row, zeros_row):
    f = pl.kernel(
        _deg_body,
        out_type=jax.ShapeDtypeStruct((2 * NP,), jnp.float32),
        mesh=_sc_mesh(),
        compiler_params=pltpu.CompilerParams(use_tc_tiling_on_sc=False),
        scratch_types=[
            pltpu.VMEM((DEG_BLK, 128), jnp.int32),
            pltpu.VMEM((128,), jnp.float32),
            pltpu.VMEM((RPT,), jnp.float32),
            pltpu.SemaphoreType.DMA,
            pltpu.VMEM_SHARED((NP,), jnp.float32),
        ],
    )
    return f(de2d, ones_row, zeros_row)


def _run_edges(h2cat, se2d, de2d, zeros_blk):
    f = pl.kernel(
        _edge_body,
        out_type=jax.ShapeDtypeStruct((2 * NP, 32), jnp.float32),
        mesh=_sc_mesh(),
        compiler_params=pltpu.CompilerParams(use_tc_tiling_on_sc=False),
        scratch_types=[
            pltpu.VMEM((BLK, ECH), jnp.int32),
            pltpu.VMEM((BLK, ECH), jnp.int32),
            pltpu.VMEM((ECH, 32), jnp.float32),
            pltpu.VMEM((ECH, 32), jnp.float32),
            pltpu.VMEM((ECH, 32), jnp.float32),
            pltpu.VMEM_SHARED((NP, 32), jnp.float32),
            pltpu.SemaphoreType.DMA,
            pltpu.SemaphoreType.DMA,
            pltpu.SemaphoreType.DMA,
        ],
    )
    return f(h2cat, se2d, de2d, zeros_blk)


def _run_a1(xp, deg2, W0):
    return pl.pallas_call(
        _a1_body,
        grid=(GRID,),
        in_specs=[
            pl.BlockSpec((ROWB, IN_DIM), lambda i: (i, 0)),
            pl.BlockSpec((2, 1, 1, ROWB), lambda i: (0, i, 0, 0)),
            pl.BlockSpec((IN_DIM, HID), lambda i: (0, 0)),
        ],
        out_specs=pl.BlockSpec((2, ROWB, 32), lambda i: (0, i, 0)),
        out_shape=jax.ShapeDtypeStruct((2, NP, 32), jnp.float32),
    )(xp, deg2, W0)


def _run_a23(z, st, g, be, deg2, W):
    return pl.pallas_call(
        _a23_body,
        grid=(GRID,),
        in_specs=[
            pl.BlockSpec((ROWB, HID), lambda i: (i, 0)),
            pl.BlockSpec((2, HID), lambda i: (0, 0)),
            pl.BlockSpec((1, HID), lambda i: (0, 0)),
            pl.BlockSpec((1, HID), lambda i: (0, 0)),
            pl.BlockSpec((2, 1, 1, ROWB), lambda i: (0, i, 0, 0)),
            pl.BlockSpec((HID, HID), lambda i: (0, 0)),
        ],
        out_specs=pl.BlockSpec((2, ROWB, 32), lambda i: (0, i, 0)),
        out_shape=jax.ShapeDtypeStruct((2, NP, 32), jnp.float32),
    )(z, st, g, be, deg2, W)


def _run_c(accs, h2pair, deg2, b):
    return pl.pallas_call(
        _c_body,
        grid=(GRID,),
        in_specs=[
            pl.BlockSpec((2, ROWB, 32), lambda i: (0, i, 0)),
            pl.BlockSpec((2, ROWB, 32), lambda i: (0, i, 0)),
            pl.BlockSpec((2, 1, 1, ROWB), lambda i: (0, i, 0, 0)),
            pl.BlockSpec((1, HID), lambda i: (0, 0)),
        ],
        out_specs=[
            pl.BlockSpec((ROWB, HID), lambda i: (i, 0)),
            pl.BlockSpec((2, HID), lambda i: (0, 0)),
        ],
        out_shape=[
            jax.ShapeDtypeStruct((NP, HID), jnp.float32),
            jax.ShapeDtypeStruct((2, HID), jnp.float32),
        ],
    )(accs, h2pair, deg2, b)


def _run_pool(z3, st3, g, be, batch3d, fcW1, fcb1, fcW2, fcb2, fcW3, fcb3):
    return pl.pallas_call(
        _pool_body,
        grid=(GRID_P,),
        in_specs=[
            pl.BlockSpec((ROWB_P, HID), lambda i: (i, 0)),
            pl.BlockSpec((2, HID), lambda i: (0, 0)),
            pl.BlockSpec((1, HID), lambda i: (0, 0)),
            pl.BlockSpec((1, HID), lambda i: (0, 0)),
            pl.BlockSpec((1, 1, ROWB_P), lambda i: (i, 0, 0)),
            pl.BlockSpec((3 * HID, HID), lambda i: (0, 0)),
            pl.BlockSpec((1, HID), lambda i: (0, 0)),
            pl.BlockSpec((HID, HID // 2), lambda i: (0, 0)),
            pl.BlockSpec((1, HID // 2), lambda i: (0, 0)),
            pl.BlockSpec((HID // 2, 1), lambda i: (0, 0)),
            pl.BlockSpec((1, 1), lambda i: (0, 0)),
        ],
        out_specs=pl.BlockSpec((G, 1), lambda i: (0, 0)),
        out_shape=jax.ShapeDtypeStruct((G, 1), jnp.float32),
        scratch_shapes=[
            pltpu.VMEM((G, HID), jnp.float32),
            pltpu.VMEM((G, HID), jnp.float32),
            pltpu.VMEM((G, 1), jnp.float32),
        ],
    )(z3, st3, g, be, batch3d, fcW1, fcb1, fcW2, fcb2, fcW3, fcb3)


def kernel(x, edge_index, batch, W0, b0, g0, be0, W1, b1, g1, be1,
           W2, b2, g2, be2, fcW1, fcb1, fcW2, fcb2, fcW3, fcb3):
    src = edge_index[0]
    dst = edge_index[1]

    padn = EP - E
    pad_src = (jnp.arange(padn, dtype=jnp.int32) * 67) % N
    pad_dst = N + (jnp.arange(padn, dtype=jnp.int32) % (NP - N))
    se2d = jnp.concatenate([src, pad_src]).reshape(EP // ECH, ECH)
    de2d = jnp.concatenate([dst, pad_dst]).reshape(EP // ECH, ECH)
    padd = EP_DEG - E
    pad_dst_deg = N + (jnp.arange(padd, dtype=jnp.int32) % (NP - N))
    ded2d = jnp.concatenate([dst, pad_dst_deg]).reshape(EP_DEG // 128, 128)
    xp = x
    batch3d = jnp.pad(batch, (0, NP - N), constant_values=G).reshape(
        GRID_P, 1, ROWB_P)
    ones_row = jnp.ones((128,), jnp.float32)
    zeros_row = jnp.zeros((RPT,), jnp.float32)
    zeros_blk = jnp.zeros((ECH, 32), jnp.float32)

    deg2 = _run_deg(ded2d, ones_row, zeros_row).reshape(2, GRID, 1, ROWB)

    h2p = _run_a1(xp, deg2, W0)
    accs = _run_edges(h2p.reshape(2 * NP, 32), se2d, de2d, zeros_blk)
    z, st = _run_c(accs.reshape(2, NP, 32), h2p, deg2, b0[None, :])

    h2p = _run_a23(z, st, g0[None, :], be0[None, :], deg2, W1)
    accs = _run_edges(h2p.reshape(2 * NP, 32), se2d, de2d, zeros_blk)
    z, st = _run_c(accs.reshape(2, NP, 32), h2p, deg2, b1[None, :])

    h2p = _run_a23(z, st, g1[None, :], be1[None, :], deg2, W2)
    accs = _run_edges(h2p.reshape(2 * NP, 32), se2d, de2d, zeros_blk)
    z, st = _run_c(accs.reshape(2, NP, 32), h2p, deg2, b2[None, :])

    out = _run_pool(z, st, g2[None, :], be2[None, :], batch3d,
                    fcW1, fcb1[None, :], fcW2, fcb2[None, :],
                    fcW3, fcb3[None, :])
    return out
